# Initial kernel scaffold; baseline (speedup 1.0000x reference)
import functools

import jax
import jax.numpy as jnp
from jax.experimental import pallas as pl
from jax.experimental.pallas import tpu as pltpu

_DELTA = 0.2
_INT_MIN = jnp.int32(-2147483648)

_ROWS = 8


def _body(t_ref, x_ref, out_ref, s_ref, *, n, num_k, inv_m):
    i = pl.program_id(0)
    tgt = t_ref[...]
    x = x_ref[...]
    col = jax.lax.broadcasted_iota(jnp.int32, (_ROWS, n), 1)
    is_t = col == tgt
    pos = jnp.sum(jnp.where(is_t, x, 0.0), axis=1)
    xm = jnp.where(is_t, jnp.float32(-1e30), x)
    bits = jax.lax.bitcast_convert_type(xm, jnp.int32)
    s = jnp.where(bits >= 0, bits, jnp.bitwise_not(bits) ^ _INT_MIN)
    s_ref[...] = s

    def step(b, prefix):
        bit = jnp.left_shift(jnp.int32(1), 31 - b)
        cand = prefix | bit
        thr = cand ^ _INT_MIN
        cnt = jnp.sum(
            jnp.where(s_ref[...] >= thr, jnp.int32(1), jnp.int32(0)),
            axis=1, keepdims=True)
        return jnp.where(cnt >= num_k, cand, prefix)

    prefix = jax.lax.fori_loop(0, 32, step,
                               jnp.zeros((_ROWS, 1), jnp.int32))
    thr = prefix ^ _INT_MIN

    s = s_ref[...]
    gt = s > thr
    eq = s == thr
    f = (1.0 + xm) ** 2
    cnt_gt = jnp.sum(jnp.where(gt, jnp.int32(1), jnp.int32(0)), axis=1)
    cnt_eq = jnp.sum(jnp.where(eq, jnp.int32(1), jnp.int32(0)), axis=1)
    sum_gt = jnp.sum(jnp.where(gt, f, 0.0), axis=1)
    sum_eq = jnp.sum(jnp.where(eq, f, 0.0), axis=1)
    need = (num_k - cnt_gt).astype(jnp.float32)
    safe_eq = jnp.maximum(cnt_eq, 1).astype(jnp.float32)
    top_sum = sum_gt + jnp.where(need > 0, sum_eq * need / safe_eq, 0.0)
    per_row = (1.0 - pos) ** 2 + (_DELTA / num_k) * top_sum
    blk = jnp.sum(per_row) * inv_m

    @pl.when(i == 0)
    def _():
        out_ref[0, 0] = 0.0
    out_ref[0, 0] += blk


def kernel(inputs, targets):
    m, n = inputs.shape
    num_k = int(0.01 * (n - 1))
    t2 = targets.astype(jnp.int32).reshape(m, 1)
    body = functools.partial(_body, n=n, num_k=num_k, inv_m=1.0 / m)
    out = pl.pallas_call(
        body,
        grid=(m // _ROWS,),
        in_specs=[
            pl.BlockSpec((_ROWS, 1), lambda i: (i, 0)),
            pl.BlockSpec((_ROWS, n), lambda i: (i, 0)),
        ],
        out_specs=pl.BlockSpec((1, 1), lambda i: (0, 0)),
        out_shape=jax.ShapeDtypeStruct((1, 1), jnp.float32),
        scratch_shapes=[pltpu.VMEM((_ROWS, n), jnp.int32)],
    )(t2, inputs)
    return out[0, 0]


# speedup vs baseline: 13.4639x; 13.4639x over previous
import functools

import jax
import jax.numpy as jnp
from jax.experimental import pallas as pl
from jax.experimental.pallas import tpu as pltpu

_DELTA = 0.2
_INT_MIN = -2147483648

_ROWS = 8


def _body(t_ref, x_ref, out_ref, s_ref, *, n, num_k, inv_m):
    i = pl.program_id(0)
    tgt = t_ref[...]
    x = x_ref[...]
    col = jax.lax.broadcasted_iota(jnp.int32, (_ROWS, n), 1)
    is_t = col == tgt
    pos = jnp.sum(jnp.where(is_t, x, 0.0), axis=1)
    xm = jnp.where(is_t, jnp.float32(-1e30), x)
    int_min = jnp.int32(_INT_MIN)
    bits = jax.lax.bitcast_convert_type(xm, jnp.int32)
    s = jnp.where(bits >= 0, bits, jnp.bitwise_not(bits) ^ int_min)
    s_ref[...] = s

    def step(b, prefix):
        bit = jnp.left_shift(jnp.int32(1), 31 - b)
        cand = prefix | bit
        thr = cand ^ _INT_MIN
        cnt = jnp.sum(
            jnp.where(s_ref[...] >= thr, jnp.int32(1), jnp.int32(0)),
            axis=1, keepdims=True)
        return jnp.where(cnt >= num_k, cand, prefix)

    prefix = jax.lax.fori_loop(0, 32, step,
                               jnp.zeros((_ROWS, 1), jnp.int32))
    thr = prefix ^ _INT_MIN

    s = s_ref[...]
    gt = s > thr
    eq = s == thr
    f = (1.0 + xm) ** 2
    cnt_gt = jnp.sum(jnp.where(gt, jnp.int32(1), jnp.int32(0)), axis=1)
    cnt_eq = jnp.sum(jnp.where(eq, jnp.int32(1), jnp.int32(0)), axis=1)
    sum_gt = jnp.sum(jnp.where(gt, f, 0.0), axis=1)
    sum_eq = jnp.sum(jnp.where(eq, f, 0.0), axis=1)
    need = (num_k - cnt_gt).astype(jnp.float32)
    safe_eq = jnp.maximum(cnt_eq, 1).astype(jnp.float32)
    top_sum = sum_gt + jnp.where(need > 0, sum_eq * need / safe_eq, 0.0)
    per_row = (1.0 - pos) ** 2 + (_DELTA / num_k) * top_sum
    blk = jnp.sum(per_row) * inv_m

    @pl.when(i == 0)
    def _():
        out_ref[...] = jnp.zeros_like(out_ref)
    out_ref[...] += blk


def kernel(inputs, targets):
    m, n = inputs.shape
    num_k = int(0.01 * (n - 1))
    t2 = targets.astype(jnp.int32).reshape(m, 1)
    body = functools.partial(_body, n=n, num_k=num_k, inv_m=1.0 / m)
    out = pl.pallas_call(
        body,
        grid=(m // _ROWS,),
        in_specs=[
            pl.BlockSpec((_ROWS, 1), lambda i: (i, 0)),
            pl.BlockSpec((_ROWS, n), lambda i: (i, 0)),
        ],
        out_specs=pl.BlockSpec((1, 1), lambda i: (0, 0)),
        out_shape=jax.ShapeDtypeStruct((1, 1), jnp.float32),
        scratch_shapes=[pltpu.VMEM((_ROWS, n), jnp.int32)],
    )(t2, inputs)
    return out[0, 0]
